# Initial kernel scaffold; baseline (speedup 1.0000x reference)
import jax
import jax.numpy as jnp
from jax.experimental import pallas as pl


def kernel(x, W1l, b1, W1r, W2l, b2, W2r, W3l, b3, W3r, edge_index):
    raise NotImplementedError("write your pallas kernel here")


# speedup vs baseline: 8.1343x; 8.1343x over previous
import functools

import jax
import jax.numpy as jnp
from jax import lax
from jax.experimental import pallas as pl
from jax.experimental.pallas import tpu as pltpu
from jax.experimental.pallas import tpu_sc as plsc

N = 10000
D = 128
E = 320000
NC = 2
NS = 16
NW = NC * NS
EPW = E // NW
BS = 80
STEPS = EPW // BS
NPAD = 10240
RPT = NPAD // NS
CPT = NPAD // NS

_mesh = plsc.VectorSubcoreMesh(core_axis_name="c", subcore_axis_name="s")


@functools.partial(
    pl.kernel,
    out_type=jax.ShapeDtypeStruct((NC, NPAD), jnp.float32),
    mesh=_mesh,
    scratch_types=[
        pltpu.VMEM((STEPS, BS), jnp.int32),
        pltpu.VMEM((BS,), jnp.float32),
        pltpu.VMEM_SHARED((NPAD,), jnp.float32),
    ],
)
def _sc_counts(dst3, zpad, out, idx_v, ones_v, cnt_sh):
    cid = lax.axis_index("c")
    sid = lax.axis_index("s")
    wid = cid * NS + sid
    pltpu.sync_copy(zpad.at[pl.ds(sid * CPT, CPT)], cnt_sh.at[pl.ds(sid * CPT, CPT)])
    for j in range(BS // 16):
        ones_v[pl.ds(j * 16, 16)] = jnp.ones((16,), jnp.float32)
    pltpu.sync_copy(dst3.at[wid], idx_v)
    plsc.subcore_barrier()

    def body(i, carry):
        pltpu.sync_copy(ones_v, cnt_sh.at[idx_v.at[i]], add=True)
        return carry

    lax.fori_loop(0, STEPS, body, 0)
    plsc.subcore_barrier()
    pltpu.sync_copy(cnt_sh.at[pl.ds(sid * CPT, CPT)], out.at[cid, pl.ds(sid * CPT, CPT)])


@functools.partial(
    pl.kernel,
    out_type=jax.ShapeDtypeStruct((NC, NPAD, D), jnp.float32),
    mesh=_mesh,
    scratch_types=[
        pltpu.VMEM((STEPS, BS), jnp.int32),
        pltpu.VMEM((STEPS, BS), jnp.int32),
        pltpu.VMEM((BS, D), jnp.float32),
        pltpu.VMEM_SHARED((NPAD, D), jnp.float32),
    ],
)
def _sc_agg(y, src3, dst3, zrows, out, src_v, dst_v, rows_v, acc_sh):
    cid = lax.axis_index("c")
    sid = lax.axis_index("s")
    wid = cid * NS + sid
    pltpu.sync_copy(zrows.at[pl.ds(sid * RPT, RPT)], acc_sh.at[pl.ds(sid * RPT, RPT)])
    pltpu.sync_copy(src3.at[wid], src_v)
    pltpu.sync_copy(dst3.at[wid], dst_v)
    plsc.subcore_barrier()

    def body(i, carry):
        pltpu.sync_copy(y.at[src_v.at[i]], rows_v)
        pltpu.sync_copy(rows_v, acc_sh.at[dst_v.at[i]], add=True)
        return carry

    lax.fori_loop(0, STEPS, body, 0)
    plsc.subcore_barrier()
    pltpu.sync_copy(acc_sh.at[pl.ds(sid * RPT, RPT)], out.at[cid, pl.ds(sid * RPT, RPT)])


RB = 1000


def _dense1_body(c0, c1, s0, s1, x, wl, bl, wr, h_out, invc_out):
    invc = 1.0 / jnp.maximum(c0[...] + c1[...], 1.0)
    mean = (s0[...] + s1[...]) * invc
    h = (jnp.dot(mean, wl[...], preferred_element_type=jnp.float32) + bl[...]
         + jnp.dot(x[...], wr[...], preferred_element_type=jnp.float32))
    h_out[...] = jnp.maximum(h, 0.0)
    invc_out[...] = invc


def _dense2_body(invc, s0, s1, h1, wl, bl, wr, w3l, h_out, y3_out):
    mean = (s0[...] + s1[...]) * invc[...]
    h = (jnp.dot(mean, wl[...], preferred_element_type=jnp.float32) + bl[...]
         + jnp.dot(h1[...], wr[...], preferred_element_type=jnp.float32))
    h = jnp.maximum(h, 0.0)
    h_out[...] = h
    y3_out[...] = jnp.dot(h, w3l[...], preferred_element_type=jnp.float32)


def _dense3_body(invc, s0, s1, h2, bl, wr, h_out):
    mean = (s0[...] + s1[...]) * invc[...]
    h = mean + bl[...] + jnp.dot(h2[...], wr[...], preferred_element_type=jnp.float32)
    h_out[...] = jnp.maximum(h, 0.0)


def _row_spec(w):
    return pl.BlockSpec((RB, w), lambda i: (i, 0))


def _full_spec(shape):
    return pl.BlockSpec(shape, lambda i: tuple(0 for _ in shape))


def kernel(x, W1l, b1, W1r, W2l, b2, W2r, W3l, b3, W3r, edge_index):
    src3 = edge_index[0].reshape(NW, STEPS, BS)
    dst3 = edge_index[1].reshape(NW, STEPS, BS)
    zpad = jnp.zeros((NPAD,), jnp.float32)
    zrows = jnp.zeros((NPAD, D), jnp.float32)

    cnt = _sc_counts(dst3, zpad)
    c0 = cnt[0, :N, None]
    c1 = cnt[1, :N, None]

    grid = (N // RB,)

    s = _sc_agg(x, src3, dst3, zrows)[:, :N]
    h1, invc = pl.pallas_call(
        _dense1_body,
        grid=grid,
        in_specs=[_row_spec(1), _row_spec(1), _row_spec(D), _row_spec(D),
                  _row_spec(D), _full_spec((D, D)), _full_spec((1, D)),
                  _full_spec((D, D))],
        out_specs=[_row_spec(D), _row_spec(1)],
        out_shape=[jax.ShapeDtypeStruct((N, D), jnp.float32),
                   jax.ShapeDtypeStruct((N, 1), jnp.float32)],
    )(c0, c1, s[0], s[1], x, W1l, b1.reshape(1, D), W1r)

    s = _sc_agg(h1, src3, dst3, zrows)[:, :N]
    h2, y3 = pl.pallas_call(
        _dense2_body,
        grid=grid,
        in_specs=[_row_spec(1), _row_spec(D), _row_spec(D), _row_spec(D),
                  _full_spec((D, 2 * D)), _full_spec((1, 2 * D)),
                  _full_spec((D, 2 * D)), _full_spec((2 * D, D))],
        out_specs=[_row_spec(2 * D), _row_spec(D)],
        out_shape=[jax.ShapeDtypeStruct((N, 2 * D), jnp.float32),
                   jax.ShapeDtypeStruct((N, D), jnp.float32)],
    )(invc, s[0], s[1], h1, W2l, b2.reshape(1, 2 * D), W2r, W3l)

    s = _sc_agg(y3, src3, dst3, zrows)[:, :N]
    h3 = pl.pallas_call(
        _dense3_body,
        grid=grid,
        in_specs=[_row_spec(1), _row_spec(D), _row_spec(D), _row_spec(2 * D),
                  _full_spec((1, D)), _full_spec((2 * D, D))],
        out_specs=_row_spec(D),
        out_shape=jax.ShapeDtypeStruct((N, D), jnp.float32),
    )(invc, s[0], s[1], h2, b3.reshape(1, D), W3r)
    return h3


# speedup vs baseline: 11.4896x; 1.4125x over previous
import functools

import jax
import jax.numpy as jnp
from jax import lax
from jax.experimental import pallas as pl
from jax.experimental.pallas import tpu as pltpu
from jax.experimental.pallas import tpu_sc as plsc

N = 10000
D = 128
E = 320000
NC = 2
NS = 16
NW = NC * NS
BS = 64
EPW = 10240
STEPS = EPW // BS
PAIRS = STEPS // 2
EPAD = EPW * NW
NPAD = 10240
RPT = NPAD // NS
CPT = NPAD // NS

_mesh = plsc.VectorSubcoreMesh(core_axis_name="c", subcore_axis_name="s")


@functools.partial(
    pl.kernel,
    out_type=jax.ShapeDtypeStruct((NC, NPAD), jnp.float32),
    mesh=_mesh,
    scratch_types=[
        pltpu.VMEM((STEPS, BS), jnp.int32),
        pltpu.VMEM((BS,), jnp.float32),
        pltpu.VMEM_SHARED((NPAD,), jnp.float32),
    ],
)
def _sc_counts(dst3, zpad, out, idx_v, ones_v, cnt_sh):
    cid = lax.axis_index("c")
    sid = lax.axis_index("s")
    wid = cid * NS + sid
    pltpu.sync_copy(zpad.at[pl.ds(sid * CPT, CPT)], cnt_sh.at[pl.ds(sid * CPT, CPT)])
    for j in range(BS // 16):
        ones_v[pl.ds(j * 16, 16)] = jnp.ones((16,), jnp.float32)
    pltpu.sync_copy(dst3.at[wid], idx_v)
    plsc.subcore_barrier()

    def body(i, carry):
        pltpu.sync_copy(ones_v, cnt_sh.at[idx_v.at[i]], add=True)
        return carry

    lax.fori_loop(0, STEPS, body, 0)
    plsc.subcore_barrier()
    pltpu.sync_copy(cnt_sh.at[pl.ds(sid * CPT, CPT)], out.at[cid, pl.ds(sid * CPT, CPT)])


HALVES = 2
HSTEPS = STEPS // HALVES
HPAIRS = HSTEPS // 2


@functools.partial(
    pl.kernel,
    out_type=jax.ShapeDtypeStruct((NC, NPAD, D), jnp.float32),
    mesh=_mesh,
    scratch_types=[
        pltpu.VMEM((HSTEPS, BS), jnp.int32),
        pltpu.VMEM((HSTEPS, BS), jnp.int32),
        pltpu.VMEM((BS, D), jnp.float32),
        pltpu.VMEM((BS, D), jnp.float32),
        pltpu.VMEM_SHARED((NPAD, D), jnp.float32),
        pltpu.SemaphoreType.DMA,
        pltpu.SemaphoreType.DMA,
    ],
)
def _sc_agg(y, src3, dst3, zrows, out, src_v, dst_v, rows0, rows1, acc_sh,
            sem0, sem1):
    cid = lax.axis_index("c")
    sid = lax.axis_index("s")
    wid = cid * NS + sid
    pltpu.sync_copy(zrows.at[pl.ds(sid * RPT, RPT)], acc_sh.at[pl.ds(sid * RPT, RPT)])
    plsc.subcore_barrier()

    def gstart(step, buf, sem):
        pltpu.async_copy(y.at[src_v.at[step]], buf, sem)

    def gwait(step, buf, sem):
        pltpu.make_async_copy(y.at[src_v.at[step]], buf, sem).wait()

    def scat(step, buf):
        pltpu.sync_copy(buf, acc_sh.at[dst_v.at[step]], add=True)

    def body(g, carry):
        s0 = 2 * g
        s1 = s0 + 1
        gstart(s1, rows1, sem1)
        gwait(s0, rows0, sem0)
        scat(s0, rows0)

        @pl.when(g < HPAIRS - 1)
        def _():
            gstart(s0 + 2, rows0, sem0)

        gwait(s1, rows1, sem1)
        scat(s1, rows1)
        return carry

    for h in range(HALVES):
        pltpu.sync_copy(src3.at[wid, pl.ds(h * HSTEPS, HSTEPS)], src_v)
        pltpu.sync_copy(dst3.at[wid, pl.ds(h * HSTEPS, HSTEPS)], dst_v)
        gstart(0, rows0, sem0)
        lax.fori_loop(0, HPAIRS, body, 0)
    plsc.subcore_barrier()
    pltpu.sync_copy(acc_sh.at[pl.ds(sid * RPT, RPT)], out.at[cid, pl.ds(sid * RPT, RPT)])


RB = 1000


def _dense1_body(c0, c1, s0, s1, x, wl, bl, wr, h_out, invc_out):
    invc = 1.0 / jnp.maximum(c0[...] + c1[...], 1.0)
    mean = (s0[...] + s1[...]) * invc
    h = (jnp.dot(mean, wl[...], preferred_element_type=jnp.float32) + bl[...]
         + jnp.dot(x[...], wr[...], preferred_element_type=jnp.float32))
    h_out[...] = jnp.maximum(h, 0.0)
    invc_out[...] = invc


def _dense2_body(invc, s0, s1, h1, wl, bl, wr, w3l, h_out, y3_out):
    mean = (s0[...] + s1[...]) * invc[...]
    h = (jnp.dot(mean, wl[...], preferred_element_type=jnp.float32) + bl[...]
         + jnp.dot(h1[...], wr[...], preferred_element_type=jnp.float32))
    h = jnp.maximum(h, 0.0)
    h_out[...] = h
    y3_out[...] = jnp.dot(h, w3l[...], preferred_element_type=jnp.float32)


def _dense3_body(invc, s0, s1, h2, bl, wr, h_out):
    mean = (s0[...] + s1[...]) * invc[...]
    h = mean + bl[...] + jnp.dot(h2[...], wr[...], preferred_element_type=jnp.float32)
    h_out[...] = jnp.maximum(h, 0.0)


def _row_spec(w):
    return pl.BlockSpec((RB, w), lambda i: (i, 0))


def _full_spec(shape):
    return pl.BlockSpec(shape, lambda i: tuple(0 for _ in shape))


def kernel(x, W1l, b1, W1r, W2l, b2, W2r, W3l, b3, W3r, edge_index):
    npad_e = EPAD - E
    pad_iota = jnp.arange(npad_e, dtype=jnp.int32)
    src3 = jnp.concatenate(
        [edge_index[0], pad_iota % N]).reshape(NW, STEPS, BS)
    dst3 = jnp.concatenate(
        [edge_index[1], N + pad_iota % (NPAD - N)]).reshape(NW, STEPS, BS)
    zpad = jnp.zeros((NPAD,), jnp.float32)
    zrows = jnp.zeros((NPAD, D), jnp.float32)

    cnt = _sc_counts(dst3, zpad)
    c0 = cnt[0, :N, None]
    c1 = cnt[1, :N, None]

    grid = (N // RB,)

    s = _sc_agg(x, src3, dst3, zrows)[:, :N]
    h1, invc = pl.pallas_call(
        _dense1_body,
        grid=grid,
        in_specs=[_row_spec(1), _row_spec(1), _row_spec(D), _row_spec(D),
                  _row_spec(D), _full_spec((D, D)), _full_spec((1, D)),
                  _full_spec((D, D))],
        out_specs=[_row_spec(D), _row_spec(1)],
        out_shape=[jax.ShapeDtypeStruct((N, D), jnp.float32),
                   jax.ShapeDtypeStruct((N, 1), jnp.float32)],
    )(c0, c1, s[0], s[1], x, W1l, b1.reshape(1, D), W1r)

    s = _sc_agg(h1, src3, dst3, zrows)[:, :N]
    h2, y3 = pl.pallas_call(
        _dense2_body,
        grid=grid,
        in_specs=[_row_spec(1), _row_spec(D), _row_spec(D), _row_spec(D),
                  _full_spec((D, 2 * D)), _full_spec((1, 2 * D)),
                  _full_spec((D, 2 * D)), _full_spec((2 * D, D))],
        out_specs=[_row_spec(2 * D), _row_spec(D)],
        out_shape=[jax.ShapeDtypeStruct((N, 2 * D), jnp.float32),
                   jax.ShapeDtypeStruct((N, D), jnp.float32)],
    )(invc, s[0], s[1], h1, W2l, b2.reshape(1, 2 * D), W2r, W3l)

    s = _sc_agg(y3, src3, dst3, zrows)[:, :N]
    h3 = pl.pallas_call(
        _dense3_body,
        grid=grid,
        in_specs=[_row_spec(1), _row_spec(D), _row_spec(D), _row_spec(2 * D),
                  _full_spec((1, D)), _full_spec((2 * D, D))],
        out_specs=_row_spec(D),
        out_shape=jax.ShapeDtypeStruct((N, D), jnp.float32),
    )(invc, s[0], s[1], h2, b3.reshape(1, D), W3r)
    return h3


# speedup vs baseline: 12.6601x; 1.1019x over previous
import functools

import jax
import jax.numpy as jnp
from jax import lax
from jax.experimental import pallas as pl
from jax.experimental.pallas import tpu as pltpu
from jax.experimental.pallas import tpu_sc as plsc

N = 10000
D = 128
E = 320000
NC = 2
NS = 16
NW = NC * NS
BS = 128
EPW = 10240
STEPS = EPW // BS
PAIRS = STEPS // 2
EPAD = EPW * NW
NPAD = 10240
RPT = NPAD // NS
CPT = NPAD // NS
HALVES = 4
HSTEPS = STEPS // HALVES
HPAIRS = HSTEPS // 2

_mesh = plsc.VectorSubcoreMesh(core_axis_name="c", subcore_axis_name="s")


@functools.partial(
    pl.kernel,
    out_type=jax.ShapeDtypeStruct((NC, NPAD), jnp.float32),
    mesh=_mesh,
    scratch_types=[
        pltpu.VMEM((HSTEPS, BS), jnp.int32),
        pltpu.VMEM((BS,), jnp.float32),
        pltpu.VMEM_SHARED((NPAD,), jnp.float32),
    ],
)
def _sc_counts(dst3, zpad, out, idx_v, ones_v, cnt_sh):
    cid = lax.axis_index("c")
    sid = lax.axis_index("s")
    wid = cid * NS + sid
    pltpu.sync_copy(zpad.at[pl.ds(sid * CPT, CPT)], cnt_sh.at[pl.ds(sid * CPT, CPT)])
    for j in range(BS // 16):
        ones_v[pl.ds(j * 16, 16)] = jnp.ones((16,), jnp.float32)
    plsc.subcore_barrier()

    def body(i, carry):
        pltpu.sync_copy(ones_v, cnt_sh.at[idx_v.at[i]], add=True)
        return carry

    for h in range(HALVES):
        pltpu.sync_copy(dst3.at[wid, h], idx_v)
        lax.fori_loop(0, HSTEPS, body, 0)
    plsc.subcore_barrier()
    pltpu.sync_copy(cnt_sh.at[pl.ds(sid * CPT, CPT)], out.at[cid, pl.ds(sid * CPT, CPT)])


@functools.partial(
    pl.kernel,
    out_type=jax.ShapeDtypeStruct((NC, NPAD, D), jnp.float32),
    mesh=_mesh,
    scratch_types=[
        pltpu.VMEM((HSTEPS, BS), jnp.int32),
        pltpu.VMEM((HSTEPS, BS), jnp.int32),
        pltpu.VMEM((BS, D), jnp.float32),
        pltpu.VMEM((BS, D), jnp.float32),
        pltpu.VMEM_SHARED((NPAD, D), jnp.float32),
        pltpu.SemaphoreType.DMA,
        pltpu.SemaphoreType.DMA,
    ],
)
def _sc_agg(y, src3, dst3, zrows, out, src_v, dst_v, rows0, rows1, acc_sh,
            sem0, sem1):
    cid = lax.axis_index("c")
    sid = lax.axis_index("s")
    wid = cid * NS + sid
    pltpu.sync_copy(zrows.at[pl.ds(sid * RPT, RPT)], acc_sh.at[pl.ds(sid * RPT, RPT)])
    plsc.subcore_barrier()

    def gstart(step, buf, sem):
        pltpu.async_copy(y.at[src_v.at[step]], buf, sem)

    def gwait(step, buf, sem):
        pltpu.make_async_copy(y.at[src_v.at[step]], buf, sem).wait()

    def scat(step, buf):
        pltpu.sync_copy(buf, acc_sh.at[dst_v.at[step]], add=True)

    def body(g, carry):
        s0 = 2 * g
        s1 = s0 + 1
        gstart(s1, rows1, sem1)
        gwait(s0, rows0, sem0)
        scat(s0, rows0)

        @pl.when(g < HPAIRS - 1)
        def _():
            gstart(s0 + 2, rows0, sem0)

        gwait(s1, rows1, sem1)
        scat(s1, rows1)
        return carry

    for h in range(HALVES):
        pltpu.sync_copy(src3.at[wid, h], src_v)
        pltpu.sync_copy(dst3.at[wid, h], dst_v)
        gstart(0, rows0, sem0)
        lax.fori_loop(0, HPAIRS, body, 0)
    plsc.subcore_barrier()
    pltpu.sync_copy(acc_sh.at[pl.ds(sid * RPT, RPT)], out.at[cid, pl.ds(sid * RPT, RPT)])


RB = 1000


def _dense1_body(c0, c1, s0, s1, x, wl, bl, wr, h_out, invc_out):
    invc = 1.0 / jnp.maximum(c0[...] + c1[...], 1.0)
    mean = (s0[...] + s1[...]) * invc
    h = (jnp.dot(mean, wl[...], preferred_element_type=jnp.float32) + bl[...]
         + jnp.dot(x[...], wr[...], preferred_element_type=jnp.float32))
    h_out[...] = jnp.maximum(h, 0.0)
    invc_out[...] = invc


def _dense2_body(invc, s0, s1, h1, wl, bl, wr, w3l, h_out, y3_out):
    mean = (s0[...] + s1[...]) * invc[...]
    h = (jnp.dot(mean, wl[...], preferred_element_type=jnp.float32) + bl[...]
         + jnp.dot(h1[...], wr[...], preferred_element_type=jnp.float32))
    h = jnp.maximum(h, 0.0)
    h_out[...] = h
    y3_out[...] = jnp.dot(h, w3l[...], preferred_element_type=jnp.float32)


def _dense3_body(invc, s0, s1, h2, bl, wr, h_out):
    mean = (s0[...] + s1[...]) * invc[...]
    h = mean + bl[...] + jnp.dot(h2[...], wr[...], preferred_element_type=jnp.float32)
    h_out[...] = jnp.maximum(h, 0.0)


def _row_spec(w):
    return pl.BlockSpec((RB, w), lambda i: (i, 0))


def _full_spec(shape):
    return pl.BlockSpec(shape, lambda i: tuple(0 for _ in shape))


def kernel(x, W1l, b1, W1r, W2l, b2, W2r, W3l, b3, W3r, edge_index):
    npad_e = EPAD - E
    pad_iota = jnp.arange(npad_e, dtype=jnp.int32)
    src3 = jnp.concatenate(
        [edge_index[0], pad_iota % N]).reshape(NW, HALVES, HSTEPS, BS)
    dst3 = jnp.concatenate(
        [edge_index[1], N + pad_iota % (NPAD - N)]).reshape(NW, HALVES, HSTEPS, BS)
    zpad = jnp.zeros((NPAD,), jnp.float32)
    zrows = jnp.zeros((NPAD, D), jnp.float32)

    cnt = _sc_counts(dst3, zpad)
    c0 = cnt[0, :N, None]
    c1 = cnt[1, :N, None]

    grid = (N // RB,)

    s = _sc_agg(x, src3, dst3, zrows)[:, :N]
    h1, invc = pl.pallas_call(
        _dense1_body,
        grid=grid,
        in_specs=[_row_spec(1), _row_spec(1), _row_spec(D), _row_spec(D),
                  _row_spec(D), _full_spec((D, D)), _full_spec((1, D)),
                  _full_spec((D, D))],
        out_specs=[_row_spec(D), _row_spec(1)],
        out_shape=[jax.ShapeDtypeStruct((N, D), jnp.float32),
                   jax.ShapeDtypeStruct((N, 1), jnp.float32)],
    )(c0, c1, s[0], s[1], x, W1l, b1.reshape(1, D), W1r)

    s = _sc_agg(h1, src3, dst3, zrows)[:, :N]
    h2, y3 = pl.pallas_call(
        _dense2_body,
        grid=grid,
        in_specs=[_row_spec(1), _row_spec(D), _row_spec(D), _row_spec(D),
                  _full_spec((D, 2 * D)), _full_spec((1, 2 * D)),
                  _full_spec((D, 2 * D)), _full_spec((2 * D, D))],
        out_specs=[_row_spec(2 * D), _row_spec(D)],
        out_shape=[jax.ShapeDtypeStruct((N, 2 * D), jnp.float32),
                   jax.ShapeDtypeStruct((N, D), jnp.float32)],
    )(invc, s[0], s[1], h1, W2l, b2.reshape(1, 2 * D), W2r, W3l)

    s = _sc_agg(y3, src3, dst3, zrows)[:, :N]
    h3 = pl.pallas_call(
        _dense3_body,
        grid=grid,
        in_specs=[_row_spec(1), _row_spec(D), _row_spec(D), _row_spec(2 * D),
                  _full_spec((1, D)), _full_spec((2 * D, D))],
        out_specs=_row_spec(D),
        out_shape=jax.ShapeDtypeStruct((N, D), jnp.float32),
    )(invc, s[0], s[1], h2, b3.reshape(1, D), W3r)
    return h3
